# Initial kernel scaffold; baseline (speedup 1.0000x reference)
import jax
import jax.numpy as jnp
from jax.experimental import pallas as pl


def kernel(target_word, context_word, negative_samples, target_table, context_table):
    raise NotImplementedError("write your pallas kernel here")


# speedup vs baseline: 5.3185x; 5.3185x over previous
import functools

import jax
import jax.numpy as jnp
from jax import lax
from jax.experimental import pallas as pl
from jax.experimental.pallas import tpu as pltpu
from jax.experimental.pallas import tpu_sc as plsc

NC = 2
NS = 16
L = 16
NW = NC * NS


def _make_sc_kernel(B, K, D, V):
    BW = B // NW
    C = 64
    NCH = BW // C
    Q = D // L
    NSTR = (C * K) // 128

    mesh = plsc.VectorSubcoreMesh(core_axis_name="c", subcore_axis_name="s")

    @functools.partial(
        pl.kernel,
        out_type=jax.ShapeDtypeStruct((B, 2 * L), jnp.float32),
        mesh=mesh,
        scratch_types=[
            pltpu.VMEM((BW,), jnp.int32),
            pltpu.VMEM((BW,), jnp.int32),
            pltpu.VMEM((BW * K // 128, 128), jnp.int32),
            pltpu.VMEM((C, D), jnp.float32),
            pltpu.VMEM((C, D), jnp.float32),
            pltpu.VMEM((C * K, D), jnp.float32),
            pltpu.VMEM((C, 2 * L), jnp.float32),
            pltpu.SemaphoreType.DMA,
        ],
        compiler_params=pltpu.CompilerParams(needs_layout_passes=False,
                                             use_tc_tiling_on_sc=False),
    )
    def sg_kernel(tw_hbm, cw_hbm, neg_hbm, tt_hbm, ct_hbm, out_hbm,
                  idx_t, idx_c, idx_n, rows_t, rows_c, rows_n, acc, sem):
        wid = lax.axis_index("s") * NC + lax.axis_index("c")
        base_w = wid * BW

        pltpu.sync_copy(tw_hbm.at[pl.ds(base_w, BW)], idx_t)
        pltpu.sync_copy(cw_hbm.at[pl.ds(base_w, BW)], idx_c)
        pltpu.sync_copy(neg_hbm.at[pl.ds(wid * (BW * K // 128), BW * K // 128)],
                        idx_n)

        def chunk_body(ci, carry):
            base = base_w + ci * C
            copies = [
                pltpu.async_copy(tt_hbm.at[idx_t.at[pl.ds(ci * C, C)]],
                                 rows_t, sem),
                pltpu.async_copy(ct_hbm.at[idx_c.at[pl.ds(ci * C, C)]],
                                 rows_c, sem),
            ]
            for j in range(NSTR):
                copies.append(
                    pltpu.async_copy(ct_hbm.at[idx_n.at[ci * NSTR + j]],
                                     rows_n.at[pl.ds(j * 128, 128)], sem))
            for cp in copies:
                cp.wait()

            lane = lax.iota(jnp.int32, L)

            def ebody(b, carry2):
                t = [rows_t[b, pl.ds(q * L, L)] for q in range(Q)]
                c = [rows_c[b, pl.ds(q * L, L)] for q in range(Q)]
                p = t[0] * c[0]
                for q in range(1, Q):
                    p = p + t[q] * c[q]
                v0 = jnp.where(lane == 0, jnp.sum(p), 0.0)
                v1 = jnp.zeros((L,), jnp.float32)
                nb = b * K
                for k in range(K):
                    n0 = rows_n[nb + k, pl.ds(0, L)]
                    s = t[0] * n0
                    for q in range(1, Q):
                        nq = rows_n[nb + k, pl.ds(q * L, L)]
                        s = s + t[q] * nq
                    col = 1 + k
                    if col < L:
                        v0 = jnp.where(lane == col, jnp.sum(s), v0)
                    else:
                        v1 = jnp.where(lane == col - L, jnp.sum(s), v1)
                acc[b, pl.ds(0, L)] = v0
                acc[b, pl.ds(L, L)] = v1
                return carry2

            lax.fori_loop(0, C, ebody, 0)
            pltpu.sync_copy(acc, out_hbm.at[pl.ds(base, C)])
            return carry

        lax.fori_loop(0, NCH, chunk_body, 0)

    return sg_kernel


def kernel(target_word, context_word, negative_samples, target_table, context_table):
    B = target_word.shape[0]
    K = negative_samples.shape[1]
    V, D = target_table.shape
    tw = target_word.astype(jnp.int32)
    cw = context_word.astype(jnp.int32)
    neg = negative_samples.astype(jnp.int32).reshape(B * K // 128, 128)
    sg = _make_sc_kernel(B, K, D, V)
    return sg(tw, cw, neg, target_table, context_table)[:, :1 + K]
